# baseline (speedup 1.0000x reference)
import functools

import jax
import jax.numpy as jnp
from jax import lax
from jax.experimental import pallas as pl
from jax.experimental.pallas import tpu as pltpu
from jax.experimental.pallas import tpu_sc as plsc

NN = 10000
NE = 320000
EMB = 128
NLAYERS = 10
NW = 32
EPW = NE // NW
KG = 40
KS = 40
NCH = EPW // KG
NCHS = EPW // KS
NP = 10240
RPT = NP // 16

def _build_gather_combine(subtract):
    nslot = 5
    ahead = 2

    @functools.partial(
        pl.kernel,
        mesh=plsc.VectorSubcoreMesh(core_axis_name="c", subcore_axis_name="s"),
        out_type=jax.ShapeDtypeStruct((NE, EMB), jnp.float32),
        scratch_types=[
            pltpu.VMEM((NCH, KG), jnp.int32),
            pltpu.VMEM((NCH, KG), jnp.int32),
        ]
        + [pltpu.VMEM((KG, EMB), jnp.float32)] * (2 * nslot)
        + [pltpu.SemaphoreType.DMA] * (2 * nslot),
    )
    def gathc(ta, tb, ia3, ib3, o, iav, ibv, *bufsem):
        bas = bufsem[0:nslot]
        bbs = bufsem[nslot:2 * nslot]
        gs = bufsem[2 * nslot:3 * nslot]
        sts = bufsem[3 * nslot:4 * nslot]
        wid = lax.axis_index("s") * 2 + lax.axis_index("c")
        base = wid * EPW
        pltpu.sync_copy(ia3.at[wid], iav)
        pltpu.sync_copy(ib3.at[wid], ibv)

        def fire(c, b):
            pltpu.async_copy(ta.at[iav.at[c]], bas[b], gs[b])
            pltpu.async_copy(tb.at[ibv.at[c]], bbs[b], gs[b])

        def combine(b):
            ba, bb = bas[b], bbs[b]

            def row(r, carry):
                for cc in range(EMB // 16):
                    sl = pl.ds(cc * 16, 16)
                    if subtract:
                        ba[r, sl] = ba[r, sl] - bb[r, sl]
                    else:
                        ba[r, sl] = ba[r, sl] + bb[r, sl]
                return carry

            lax.fori_loop(0, KG, row, 0, unroll=4)

        for c0 in range(ahead):
            fire(c0, c0)

        def step(t, carry):
            for b in range(nslot):
                c = nslot * t + b
                bp = (b + ahead) % nslot

                @pl.when(c + ahead < NCH)
                def _():
                    @pl.when(c + ahead >= nslot)
                    def _():
                        pltpu.make_async_copy(
                            bas[bp], o.at[pl.ds(base, KG)], sts[bp]).wait()
                    fire(c + ahead, bp)

                pltpu.make_async_copy(ta.at[iav.at[0]], bas[b], gs[b]).wait()
                pltpu.make_async_copy(tb.at[ibv.at[0]], bbs[b], gs[b]).wait()
                combine(b)
                pltpu.async_copy(bas[b], o.at[pl.ds(base + c * KG, KG)], sts[b])
            return carry

        lax.fori_loop(0, NCH // nslot, step, 0)
        for b in range(nslot):
            pltpu.make_async_copy(bas[b], o.at[pl.ds(base, KG)], sts[b]).wait()

    return gathc


def _build_scatter():
    nslot = 2
    ahead = 1

    @functools.partial(
        pl.kernel,
        mesh=plsc.VectorSubcoreMesh(core_axis_name="c", subcore_axis_name="s"),
        out_type=jax.ShapeDtypeStruct((2 * NP, EMB), jnp.float32),
        scratch_types=[
            pltpu.VMEM((NCHS, KS), jnp.int32),
            pltpu.VMEM_SHARED((NP, EMB), jnp.float32),
        ]
        + [pltpu.VMEM((KS, EMB), jnp.float32)] * nslot
        + [pltpu.SemaphoreType.DMA] * (2 * nslot),
    )
    def scatter(vals, idx3, zeros, out, idxv, acc, *bufsem):
        bufs = bufsem[0:nslot]
        rds = bufsem[nslot:2 * nslot]
        scs = bufsem[2 * nslot:3 * nslot]
        ci = lax.axis_index("c")
        si = lax.axis_index("s")
        wid = si * 2 + ci
        base = wid * EPW
        pltpu.sync_copy(zeros.at[pl.ds(si * RPT, RPT)],
                        acc.at[pl.ds(si * RPT, RPT)])
        pltpu.sync_copy(idx3.at[wid], idxv)
        plsc.subcore_barrier()

        def fire_read(c, b):
            pltpu.async_copy(vals.at[pl.ds(base + c * KS, KS)], bufs[b], rds[b])

        for c0 in range(ahead):
            fire_read(c0, c0)

        def step(t, carry):
            for b in range(nslot):
                c = nslot * t + b
                bp = (b + ahead) % nslot

                @pl.when(c + ahead < NCHS)
                def _():
                    @pl.when(c + ahead >= nslot)
                    def _():
                        pltpu.make_async_copy(
                            bufs[bp], acc.at[idxv.at[0]], scs[bp]).wait()
                    fire_read(c + ahead, bp)

                pltpu.make_async_copy(
                    vals.at[pl.ds(base, KS)], bufs[b], rds[b]).wait()
                pltpu.async_copy(bufs[b], acc.at[idxv.at[c]], scs[b], add=True)
            return carry

        lax.fori_loop(0, NCHS // nslot, step, 0)
        for b in range(nslot):
            pltpu.make_async_copy(bufs[b], acc.at[idxv.at[0]], scs[b]).wait()
        plsc.subcore_barrier()
        pltpu.sync_copy(acc.at[pl.ds(si * RPT, RPT)],
                        out.at[pl.ds(ci * NP + si * RPT, RPT)])

    return scatter


_SC_CACHE = {}


def _sc_kernels():
    if not _SC_CACHE:
        _SC_CACHE["gadd"] = _build_gather_combine(subtract=False)
        _SC_CACHE["gsub"] = _build_gather_combine(subtract=True)
        _SC_CACHE["scat"] = _build_scatter()
    return _SC_CACHE["gadd"], _SC_CACHE["gsub"], _SC_CACHE["scat"]


def _ln(h, g, b):
    mu = jnp.mean(h, axis=-1, keepdims=True)
    var = jnp.mean((h - mu) * (h - mu), axis=-1, keepdims=True)
    return (h - mu) * lax.rsqrt(var + 1e-5) * g + b


def _dot(a, b):
    return jnp.dot(a, b, preferred_element_type=jnp.float32)


BLK_E = 2000
BLK_N = 2000

_row = lambda i: (i, 0)
_fix = lambda i: (0, 0)


def _wspec():
    return pl.BlockSpec((EMB, EMB), _fix)


def _bspec():
    return pl.BlockSpec((1, EMB), _fix)


def _edge_body(e, g, w1, b1, w2, b2, w3, b3, lg, lb, o):
    x = e[...]
    h = _dot(x, w1[...]) + g[...] + b1[...]
    h = jnp.maximum(h, 0.0)
    h = jnp.maximum(_dot(h, w2[...]) + b2[...], 0.0)
    h = _dot(h, w3[...]) + b3[...]
    o[...] = x + _ln(h, lg[...], lb[...])


_EDGE_CALL = pl.pallas_call(
    _edge_body,
    grid=(NE // BLK_E,),
    in_specs=[pl.BlockSpec((BLK_E, EMB), _row)] * 2
    + [_wspec(), _bspec(), _wspec(), _bspec(), _wspec(), _bspec(), _bspec(), _bspec()],
    out_specs=pl.BlockSpec((BLK_E, EMB), _row),
    out_shape=jax.ShapeDtypeStruct((NE, EMB), jnp.float32),
)


def _node_body_emit(p, n, wa, wb, b1, w2, b2, w3, b3, lg, lb, wr, ws,
                    on, ob, oc):
    agg = p[0] + p[1]
    x = n[...]
    h = _dot(agg, wa[...]) + _dot(x, wb[...]) + b1[...]
    h = jnp.maximum(h, 0.0)
    h = jnp.maximum(_dot(h, w2[...]) + b2[...], 0.0)
    h = _dot(h, w3[...]) + b3[...]
    nn_ = x + _ln(h, lg[...], lb[...])
    on[...] = nn_
    ob[...] = _dot(nn_, wr[...])
    oc[...] = _dot(nn_, ws[...])


def _node_body_last(p, n, wa, wb, b1, w2, b2, w3, b3, lg, lb, on):
    agg = p[0] + p[1]
    x = n[...]
    h = _dot(agg, wa[...]) + _dot(x, wb[...]) + b1[...]
    h = jnp.maximum(h, 0.0)
    h = jnp.maximum(_dot(h, w2[...]) + b2[...], 0.0)
    h = _dot(h, w3[...]) + b3[...]
    on[...] = x + _ln(h, lg[...], lb[...])


_node_in_specs = [
    pl.BlockSpec((2, BLK_N, EMB), lambda i: (0, i, 0)),
    pl.BlockSpec((BLK_N, EMB), _row),
    _wspec(), _wspec(), _bspec(), _wspec(), _bspec(), _wspec(), _bspec(),
    _bspec(), _bspec(),
]

_NODE_CALL_EMIT = pl.pallas_call(
    _node_body_emit,
    grid=(NN // BLK_N,),
    in_specs=_node_in_specs + [_wspec(), _wspec()],
    out_specs=[pl.BlockSpec((BLK_N, EMB), _row)] * 3,
    out_shape=[jax.ShapeDtypeStruct((NN, EMB), jnp.float32)] * 3,
)

_NODE_CALL_LAST = pl.pallas_call(
    _node_body_last,
    grid=(NN // BLK_N,),
    in_specs=_node_in_specs,
    out_specs=pl.BlockSpec((BLK_N, EMB), _row),
    out_shape=jax.ShapeDtypeStruct((NN, EMB), jnp.float32),
)


def _enc_node_body(x, w1, b1, w2, b2, w3, b3, lg, lb, wr, ws, on, ob, oc):
    h = jnp.maximum(_dot(x[...], w1[...]) + b1[...], 0.0)
    h = jnp.maximum(_dot(h, w2[...]) + b2[...], 0.0)
    h = _dot(h, w3[...]) + b3[...]
    nn_ = _ln(h, lg[...], lb[...])
    on[...] = nn_
    ob[...] = _dot(nn_, wr[...])
    oc[...] = _dot(nn_, ws[...])


_ENC_NODE_CALL = pl.pallas_call(
    _enc_node_body,
    grid=(NN // BLK_N,),
    in_specs=[pl.BlockSpec((BLK_N, 24), _row),
              pl.BlockSpec((24, EMB), _fix), _bspec(),
              _wspec(), _bspec(), _wspec(), _bspec(), _bspec(), _bspec(),
              _wspec(), _wspec()],
    out_specs=[pl.BlockSpec((BLK_N, EMB), _row)] * 3,
    out_shape=[jax.ShapeDtypeStruct((NN, EMB), jnp.float32)] * 3,
)


def _enc_edge_body(rel_ref, w1p, w1d, b1, w2, b2, w3, b3, lg, lb, o):
    rel = rel_ref[...]
    dist = jnp.sqrt(jnp.sum(rel * rel, axis=-1, keepdims=True))
    h = _dot(rel, w1p[...]) + dist * w1d[...] + b1[...]
    h = jnp.maximum(h, 0.0)
    h = jnp.maximum(_dot(h, w2[...]) + b2[...], 0.0)
    h = _dot(h, w3[...]) + b3[...]
    o[...] = _ln(h, lg[...], lb[...])


_ENC_EDGE_CALL = pl.pallas_call(
    _enc_edge_body,
    grid=(NE // BLK_E,),
    in_specs=[pl.BlockSpec((BLK_E, EMB), _row)]
    + [_wspec(), _bspec(), _bspec(),
       _wspec(), _bspec(), _wspec(), _bspec(), _bspec(), _bspec()],
    out_specs=pl.BlockSpec((BLK_E, EMB), _row),
    out_shape=jax.ShapeDtypeStruct((NE, EMB), jnp.float32),
)


def _dec_body(n, w1, b1, w2, b2, w3, b3, o):
    h = jnp.maximum(_dot(n[...], w1[...]) + b1[...], 0.0)
    h = jnp.maximum(_dot(h, w2[...]) + b2[...], 0.0)
    o[...] = _dot(h, w3[...]) + b3[...]


_DEC_CALL = pl.pallas_call(
    _dec_body,
    grid=(NN // BLK_N,),
    in_specs=[pl.BlockSpec((BLK_N, EMB), _row),
              _wspec(), _bspec(), _wspec(), _bspec(), _wspec(), _bspec()],
    out_specs=pl.BlockSpec((BLK_N, EMB), _row),
    out_shape=jax.ShapeDtypeStruct((NN, EMB), jnp.float32),
)


def _r(b):
    return b.reshape(1, EMB)


def kernel(velocities, positions, params, materials, neighbor_idxs):
    recv = neighbor_idxs[:, 0].astype(jnp.int32)
    perm = jnp.argsort(recv)
    recv = recv[perm]
    send = neighbor_idxs[perm, 1].astype(jnp.int32)

    Wm, bm = params["mat_enc"]
    (w1, b1), (w2, b2), (w3, b3) = params["node_enc"]
    enc_w1 = jnp.concatenate([w1[:15], Wm @ w1[15:]], axis=0)
    enc_b1 = b1 + bm @ w1[15:]
    lng0, lnb0 = params["node_enc_ln"]

    (ew1, eb1), (ew2, eb2), (ew3, eb3) = params["edge_enc"]
    ew1p = jnp.zeros((EMB, EMB), jnp.float32).at[:3].set(ew1[:3])
    ew1d = ew1[3:4]
    elng, elnb = params["edge_enc_ln"]

    pe = []
    for i in range(NLAYERS):
        (a1, c1), (a2, c2), (a3, c3) = params["proc_edge"][i]
        pe.append((a1[:EMB], a1[EMB:2 * EMB], a1[2 * EMB:], c1, a2, c2, a3, c3))
    pn = []
    for i in range(NLAYERS):
        (a1, c1), (a2, c2), (a3, c3) = params["proc_node"][i]
        pn.append((a1[:EMB], a1[EMB:], c1, a2, c2, a3, c3))

    x_feat = jnp.concatenate(
        [velocities.reshape(NN, -1),
         jax.nn.one_hot(materials, 9, dtype=jnp.float32)], axis=1)
    pos128 = jnp.zeros((NN, EMB), jnp.float32).at[:, :3].set(positions)
    recv3 = recv.reshape(NW, NCH, KG)
    send3 = send.reshape(NW, NCH, KG)
    recv3s = recv.reshape(NW, NCHS, KS)
    zeros = jnp.zeros((NP, EMB), jnp.float32)

    gadd, gsub, scat = _sc_kernels()

    rel = gsub(pos128, pos128, recv3, send3)
    edges = _ENC_EDGE_CALL(rel, ew1p, ew1d, _r(eb1), ew2, _r(eb2),
                           ew3, _r(eb3), _r(elng), _r(elnb))
    nodes, tb, tc = _ENC_NODE_CALL(x_feat, enc_w1, _r(enc_b1), w2, _r(b2),
                                   w3, _r(b3), _r(lng0), _r(lnb0),
                                   pe[0][1], pe[0][2])

    for i in range(NLAYERS):
        w1e, _, _, c1, a2, c2, a3, c3 = pe[i]
        lg, lb = params["proc_edge_ln"][i]
        g = gadd(tb, tc, recv3, send3)
        edges = _EDGE_CALL(edges, g, w1e, _r(c1), a2, _r(c2),
                           a3, _r(c3), _r(lg), _r(lb))
        parts = scat(edges, recv3s, zeros).reshape(2, NP, EMB)[:, :NN]
        na, nb, d1, n2, d2, n3, d3 = pn[i]
        nlg, nlb = params["proc_node_ln"][i]
        if i < NLAYERS - 1:
            nodes, tb, tc = _NODE_CALL_EMIT(
                parts, nodes, na, nb, _r(d1), n2, _r(d2), n3, _r(d3),
                _r(nlg), _r(nlb), pe[i + 1][1], pe[i + 1][2])
        else:
            nodes = _NODE_CALL_LAST(
                parts, nodes, na, nb, _r(d1), n2, _r(d2), n3, _r(d3),
                _r(nlg), _r(nlb))

    (dw1, db1), (dw2, db2), (dw3, db3) = params["dec"]
    dw3p = jnp.zeros((EMB, EMB), jnp.float32).at[:, :3].set(dw3)
    db3p = jnp.zeros((EMB,), jnp.float32).at[:3].set(db3)
    out = _DEC_CALL(nodes, dw1, _r(db1), dw2, _r(db2), dw3p, _r(db3p))
    return out[:, :3]


# speedup vs baseline: 1.5091x; 1.5091x over previous
import functools

import jax
import jax.numpy as jnp
from jax import lax
from jax.experimental import pallas as pl
from jax.experimental.pallas import tpu as pltpu
from jax.experimental.pallas import tpu_sc as plsc

NN = 10000
NE = 320000
NEH = NE // 2
EMB = 128
NLAYERS = 10
NW = 32
KG = 40
KS = 40
NP = 10240
RPT = NP // 16

def _build_gather_combine(subtract, ne):
    EPW = ne // NW
    NCH = EPW // KG
    nslot = 5
    ahead = 2
    assert NCH % nslot == 0

    @functools.partial(
        pl.kernel,
        mesh=plsc.VectorSubcoreMesh(core_axis_name="c", subcore_axis_name="s"),
        out_type=jax.ShapeDtypeStruct((ne, EMB), jnp.float32),
        scratch_types=[
            pltpu.VMEM((NCH, KG), jnp.int32),
            pltpu.VMEM((NCH, KG), jnp.int32),
        ]
        + [pltpu.VMEM((KG, EMB), jnp.float32)] * (2 * nslot)
        + [pltpu.SemaphoreType.DMA] * (2 * nslot),
    )
    def gathc(ta, tb, ia3, ib3, o, iav, ibv, *bufsem):
        bas = bufsem[0:nslot]
        bbs = bufsem[nslot:2 * nslot]
        gs = bufsem[2 * nslot:3 * nslot]
        sts = bufsem[3 * nslot:4 * nslot]
        wid = lax.axis_index("s") * 2 + lax.axis_index("c")
        base = wid * EPW
        pltpu.sync_copy(ia3.at[wid], iav)
        pltpu.sync_copy(ib3.at[wid], ibv)

        def fire(c, b):
            pltpu.async_copy(ta.at[iav.at[c]], bas[b], gs[b])
            pltpu.async_copy(tb.at[ibv.at[c]], bbs[b], gs[b])

        def combine(b):
            ba, bb = bas[b], bbs[b]

            def row(r, carry):
                for cc in range(EMB // 16):
                    sl = pl.ds(cc * 16, 16)
                    if subtract:
                        ba[r, sl] = ba[r, sl] - bb[r, sl]
                    else:
                        ba[r, sl] = ba[r, sl] + bb[r, sl]
                return carry

            lax.fori_loop(0, KG, row, 0, unroll=4)

        for c0 in range(ahead):
            fire(c0, c0)

        def step(t, carry):
            for b in range(nslot):
                c = nslot * t + b
                bp = (b + ahead) % nslot

                @pl.when(c + ahead < NCH)
                def _():
                    @pl.when(c + ahead >= nslot)
                    def _():
                        pltpu.make_async_copy(
                            bas[bp], o.at[pl.ds(base, KG)], sts[bp]).wait()
                    fire(c + ahead, bp)

                pltpu.make_async_copy(ta.at[iav.at[0]], bas[b], gs[b]).wait()
                pltpu.make_async_copy(tb.at[ibv.at[0]], bbs[b], gs[b]).wait()
                combine(b)
                pltpu.async_copy(bas[b], o.at[pl.ds(base + c * KG, KG)], sts[b])
            return carry

        lax.fori_loop(0, NCH // nslot, step, 0)
        for b in range(nslot):
            pltpu.make_async_copy(bas[b], o.at[pl.ds(base, KG)], sts[b]).wait()

    return gathc


def _build_scatter(ne):
    EPW = ne // NW
    NCHS = EPW // KS
    nslot = 5
    ahead = 1
    assert NCHS % nslot == 0

    @functools.partial(
        pl.kernel,
        mesh=plsc.VectorSubcoreMesh(core_axis_name="c", subcore_axis_name="s"),
        out_type=jax.ShapeDtypeStruct((2 * NP, EMB), jnp.float32),
        scratch_types=[
            pltpu.VMEM((NCHS, KS), jnp.int32),
            pltpu.VMEM_SHARED((NP, EMB), jnp.float32),
        ]
        + [pltpu.VMEM((KS, EMB), jnp.float32)] * nslot
        + [pltpu.SemaphoreType.DMA] * (2 * nslot),
    )
    def scatter(vals, idx3, init, out, idxv, acc, *bufsem):
        bufs = bufsem[0:nslot]
        rds = bufsem[nslot:2 * nslot]
        scs = bufsem[2 * nslot:3 * nslot]
        ci = lax.axis_index("c")
        si = lax.axis_index("s")
        wid = si * 2 + ci
        base = wid * EPW
        pltpu.sync_copy(init.at[pl.ds(ci * NP + si * RPT, RPT)],
                        acc.at[pl.ds(si * RPT, RPT)])
        pltpu.sync_copy(idx3.at[wid], idxv)
        plsc.subcore_barrier()

        def fire_read(c, b):
            pltpu.async_copy(vals.at[pl.ds(base + c * KS, KS)], bufs[b], rds[b])

        for c0 in range(ahead):
            fire_read(c0, c0)

        def step(t, carry):
            for b in range(nslot):
                c = nslot * t + b
                bp = (b + ahead) % nslot

                @pl.when(c + ahead < NCHS)
                def _():
                    @pl.when(c + ahead >= nslot)
                    def _():
                        pltpu.make_async_copy(
                            bufs[bp], acc.at[idxv.at[0]], scs[bp]).wait()
                    fire_read(c + ahead, bp)

                pltpu.make_async_copy(
                    vals.at[pl.ds(base, KS)], bufs[b], rds[b]).wait()
                pltpu.async_copy(bufs[b], acc.at[idxv.at[c]], scs[b], add=True)
            return carry

        lax.fori_loop(0, NCHS // nslot, step, 0)
        for b in range(nslot):
            pltpu.make_async_copy(bufs[b], acc.at[idxv.at[0]], scs[b]).wait()
        plsc.subcore_barrier()
        pltpu.sync_copy(acc.at[pl.ds(si * RPT, RPT)],
                        out.at[pl.ds(ci * NP + si * RPT, RPT)])

    return scatter


_SC_CACHE = {}


def _sc_kernels():
    if not _SC_CACHE:
        _SC_CACHE["gadd"] = _build_gather_combine(subtract=False, ne=NEH)
        _SC_CACHE["gsub"] = _build_gather_combine(subtract=True, ne=NE)
        _SC_CACHE["scat"] = _build_scatter(ne=NEH)
    return _SC_CACHE["gadd"], _SC_CACHE["gsub"], _SC_CACHE["scat"]


def _ln(h, g, b):
    mu = jnp.mean(h, axis=-1, keepdims=True)
    var = jnp.mean((h - mu) * (h - mu), axis=-1, keepdims=True)
    return (h - mu) * lax.rsqrt(var + 1e-5) * g + b


def _dot(a, b):
    return jnp.dot(a, b, preferred_element_type=jnp.float32)


BLK_E = 2000
BLK_N = 2000

_row = lambda i: (i, 0)
_fix = lambda i: (0, 0)


def _wspec():
    return pl.BlockSpec((EMB, EMB), _fix)


def _bspec():
    return pl.BlockSpec((1, EMB), _fix)


def _edge_body(e, g, w1, b1, w2, b2, w3, b3, lg, lb, o):
    x = e[...]
    h = _dot(x, w1[...]) + g[...] + b1[...]
    h = jnp.maximum(h, 0.0)
    h = jnp.maximum(_dot(h, w2[...]) + b2[...], 0.0)
    h = _dot(h, w3[...]) + b3[...]
    o[...] = x + _ln(h, lg[...], lb[...])


_EDGE_CALL = pl.pallas_call(
    _edge_body,
    grid=(NEH // BLK_E,),
    in_specs=[pl.BlockSpec((BLK_E, EMB), _row)] * 2
    + [_wspec(), _bspec(), _wspec(), _bspec(), _wspec(), _bspec(), _bspec(), _bspec()],
    out_specs=pl.BlockSpec((BLK_E, EMB), _row),
    out_shape=jax.ShapeDtypeStruct((NEH, EMB), jnp.float32),
)


def _node_body_emit(p, n, wa, wb, b1, w2, b2, w3, b3, lg, lb, wr, ws,
                    on, ob, oc):
    agg = p[0] + p[1]
    x = n[...]
    h = _dot(agg, wa[...]) + _dot(x, wb[...]) + b1[...]
    h = jnp.maximum(h, 0.0)
    h = jnp.maximum(_dot(h, w2[...]) + b2[...], 0.0)
    h = _dot(h, w3[...]) + b3[...]
    nn_ = x + _ln(h, lg[...], lb[...])
    on[...] = nn_
    ob[...] = _dot(nn_, wr[...])
    oc[...] = _dot(nn_, ws[...])


def _node_body_last(p, n, wa, wb, b1, w2, b2, w3, b3, lg, lb, on):
    agg = p[0] + p[1]
    x = n[...]
    h = _dot(agg, wa[...]) + _dot(x, wb[...]) + b1[...]
    h = jnp.maximum(h, 0.0)
    h = jnp.maximum(_dot(h, w2[...]) + b2[...], 0.0)
    h = _dot(h, w3[...]) + b3[...]
    on[...] = x + _ln(h, lg[...], lb[...])


_node_in_specs = [
    pl.BlockSpec((2, BLK_N, EMB), lambda i: (0, i, 0)),
    pl.BlockSpec((BLK_N, EMB), _row),
    _wspec(), _wspec(), _bspec(), _wspec(), _bspec(), _wspec(), _bspec(),
    _bspec(), _bspec(),
]

_NODE_CALL_EMIT = pl.pallas_call(
    _node_body_emit,
    grid=(NN // BLK_N,),
    in_specs=_node_in_specs + [_wspec(), _wspec()],
    out_specs=[pl.BlockSpec((BLK_N, EMB), _row)] * 3,
    out_shape=[jax.ShapeDtypeStruct((NN, EMB), jnp.float32)] * 3,
)

_NODE_CALL_LAST = pl.pallas_call(
    _node_body_last,
    grid=(NN // BLK_N,),
    in_specs=_node_in_specs,
    out_specs=pl.BlockSpec((BLK_N, EMB), _row),
    out_shape=jax.ShapeDtypeStruct((NN, EMB), jnp.float32),
)


def _enc_node_body(x, w1, b1, w2, b2, w3, b3, lg, lb, wr, ws, on, ob, oc):
    h = jnp.maximum(_dot(x[...], w1[...]) + b1[...], 0.0)
    h = jnp.maximum(_dot(h, w2[...]) + b2[...], 0.0)
    h = _dot(h, w3[...]) + b3[...]
    nn_ = _ln(h, lg[...], lb[...])
    on[...] = nn_
    ob[...] = _dot(nn_, wr[...])
    oc[...] = _dot(nn_, ws[...])


_ENC_NODE_CALL = pl.pallas_call(
    _enc_node_body,
    grid=(NN // BLK_N,),
    in_specs=[pl.BlockSpec((BLK_N, 24), _row),
              pl.BlockSpec((24, EMB), _fix), _bspec(),
              _wspec(), _bspec(), _wspec(), _bspec(), _bspec(), _bspec(),
              _wspec(), _wspec()],
    out_specs=[pl.BlockSpec((BLK_N, EMB), _row)] * 3,
    out_shape=[jax.ShapeDtypeStruct((NN, EMB), jnp.float32)] * 3,
)


def _enc_edge_body(rel_ref, w1p, w1d, b1, w2, b2, w3, b3, lg, lb, o):
    rel = rel_ref[...]
    dist = jnp.sqrt(jnp.sum(rel * rel, axis=-1, keepdims=True))
    h = _dot(rel, w1p[...]) + dist * w1d[...] + b1[...]
    h = jnp.maximum(h, 0.0)
    h = jnp.maximum(_dot(h, w2[...]) + b2[...], 0.0)
    h = _dot(h, w3[...]) + b3[...]
    o[...] = _ln(h, lg[...], lb[...])


_ENC_EDGE_CALL = pl.pallas_call(
    _enc_edge_body,
    grid=(NE // BLK_E,),
    in_specs=[pl.BlockSpec((BLK_E, EMB), _row)]
    + [_wspec(), _bspec(), _bspec(),
       _wspec(), _bspec(), _wspec(), _bspec(), _bspec(), _bspec()],
    out_specs=pl.BlockSpec((BLK_E, EMB), _row),
    out_shape=jax.ShapeDtypeStruct((NE, EMB), jnp.float32),
)


def _dec_body(n, w1, b1, w2, b2, w3, b3, o):
    h = jnp.maximum(_dot(n[...], w1[...]) + b1[...], 0.0)
    h = jnp.maximum(_dot(h, w2[...]) + b2[...], 0.0)
    o[...] = _dot(h, w3[...]) + b3[...]


_DEC_CALL = pl.pallas_call(
    _dec_body,
    grid=(NN // BLK_N,),
    in_specs=[pl.BlockSpec((BLK_N, EMB), _row),
              _wspec(), _bspec(), _wspec(), _bspec(), _wspec(), _bspec()],
    out_specs=pl.BlockSpec((BLK_N, EMB), _row),
    out_shape=jax.ShapeDtypeStruct((NN, EMB), jnp.float32),
)


def _r(b):
    return b.reshape(1, EMB)


def kernel(velocities, positions, params, materials, neighbor_idxs):
    recv = neighbor_idxs[:, 0].astype(jnp.int32)
    send = neighbor_idxs[:, 1].astype(jnp.int32)

    Wm, bm = params["mat_enc"]
    (w1, b1), (w2, b2), (w3, b3) = params["node_enc"]
    enc_w1 = jnp.concatenate([w1[:15], Wm @ w1[15:]], axis=0)
    enc_b1 = b1 + bm @ w1[15:]
    lng0, lnb0 = params["node_enc_ln"]

    (ew1, eb1), (ew2, eb2), (ew3, eb3) = params["edge_enc"]
    ew1p = jnp.zeros((EMB, EMB), jnp.float32).at[:3].set(ew1[:3])
    ew1d = ew1[3:4]
    elng, elnb = params["edge_enc_ln"]

    pe = []
    for i in range(NLAYERS):
        (a1, c1), (a2, c2), (a3, c3) = params["proc_edge"][i]
        pe.append((a1[:EMB], a1[EMB:2 * EMB], a1[2 * EMB:], c1, a2, c2, a3, c3))
    pn = []
    for i in range(NLAYERS):
        (a1, c1), (a2, c2), (a3, c3) = params["proc_node"][i]
        pn.append((a1[:EMB], a1[EMB:], c1, a2, c2, a3, c3))

    x_feat = jnp.concatenate(
        [velocities.reshape(NN, -1),
         jax.nn.one_hot(materials, 9, dtype=jnp.float32)], axis=1)
    pos128 = jnp.zeros((NN, EMB), jnp.float32).at[:, :3].set(positions)
    nchf = (NE // NW) // KG
    nchh = (NEH // NW) // KG
    recv3 = recv.reshape(NW, nchf, KG)
    send3 = send.reshape(NW, nchf, KG)
    recv3a = recv[:NEH].reshape(NW, nchh, KG)
    recv3b = recv[NEH:].reshape(NW, nchh, KG)
    send3a = send[:NEH].reshape(NW, nchh, KG)
    send3b = send[NEH:].reshape(NW, nchh, KG)
    zeros = jnp.zeros((2 * NP, EMB), jnp.float32)

    gadd, gsub, scat = _sc_kernels()

    rel = gsub(pos128, pos128, recv3, send3)
    edges = _ENC_EDGE_CALL(rel, ew1p, ew1d, _r(eb1), ew2, _r(eb2),
                           ew3, _r(eb3), _r(elng), _r(elnb))
    ea, eb = edges[:NEH], edges[NEH:]
    nodes, tb, tc = _ENC_NODE_CALL(x_feat, enc_w1, _r(enc_b1), w2, _r(b2),
                                   w3, _r(b3), _r(lng0), _r(lnb0),
                                   pe[0][1], pe[0][2])

    for i in range(NLAYERS):
        w1e, _, _, c1, a2, c2, a3, c3 = pe[i]
        lg, lb = params["proc_edge_ln"][i]
        ga = gadd(tb, tc, recv3a, send3a)
        gb = gadd(tb, tc, recv3b, send3b)
        ea = _EDGE_CALL(ea, ga, w1e, _r(c1), a2, _r(c2),
                        a3, _r(c3), _r(lg), _r(lb))
        pa = scat(ea, recv3a, zeros)
        eb = _EDGE_CALL(eb, gb, w1e, _r(c1), a2, _r(c2),
                        a3, _r(c3), _r(lg), _r(lb))
        parts = scat(eb, recv3b, pa).reshape(2, NP, EMB)[:, :NN]
        na, nb, d1, n2, d2, n3, d3 = pn[i]
        nlg, nlb = params["proc_node_ln"][i]
        if i < NLAYERS - 1:
            nodes, tb, tc = _NODE_CALL_EMIT(
                parts, nodes, na, nb, _r(d1), n2, _r(d2), n3, _r(d3),
                _r(nlg), _r(nlb), pe[i + 1][1], pe[i + 1][2])
        else:
            nodes = _NODE_CALL_LAST(
                parts, nodes, na, nb, _r(d1), n2, _r(d2), n3, _r(d3),
                _r(nlg), _r(nlb))

    (dw1, db1), (dw2, db2), (dw3, db3) = params["dec"]
    dw3p = jnp.zeros((EMB, EMB), jnp.float32).at[:, :3].set(dw3)
    db3p = jnp.zeros((EMB,), jnp.float32).at[:3].set(db3)
    out = _DEC_CALL(nodes, dw1, _r(db1), dw2, _r(db2), dw3p, _r(db3p))
    return out[:, :3]


# speedup vs baseline: 1.5462x; 1.0246x over previous
import functools

import jax
import jax.numpy as jnp
from jax import lax
from jax.experimental import pallas as pl
from jax.experimental.pallas import tpu as pltpu
from jax.experimental.pallas import tpu_sc as plsc

NN = 10000
NE = 320000
NEH = NE // 2
EMB = 128
NLAYERS = 10
NW = 32
KG = 40
KS = 40
NP = 10240
RPT = NP // 16

def _build_gather_combine(subtract, ne):
    EPW = ne // NW
    NCH = EPW // KG
    nslot = 5
    ahead = 2
    assert NCH % nslot == 0

    @functools.partial(
        pl.kernel,
        mesh=plsc.VectorSubcoreMesh(core_axis_name="c", subcore_axis_name="s"),
        out_type=jax.ShapeDtypeStruct((ne, EMB), jnp.float32),
        scratch_types=[
            pltpu.VMEM((NCH, KG), jnp.int32),
            pltpu.VMEM((NCH, KG), jnp.int32),
        ]
        + [pltpu.VMEM((KG, EMB), jnp.float32)] * (2 * nslot)
        + [pltpu.SemaphoreType.DMA] * (2 * nslot),
    )
    def gathc(ta, tb, ia3, ib3, o, iav, ibv, *bufsem):
        bas = bufsem[0:nslot]
        bbs = bufsem[nslot:2 * nslot]
        gs = bufsem[2 * nslot:3 * nslot]
        sts = bufsem[3 * nslot:4 * nslot]
        wid = lax.axis_index("s") * 2 + lax.axis_index("c")
        base = wid * EPW
        pltpu.sync_copy(ia3.at[wid], iav)
        pltpu.sync_copy(ib3.at[wid], ibv)

        def fire(c, b):
            pltpu.async_copy(ta.at[iav.at[c]], bas[b], gs[b])
            pltpu.async_copy(tb.at[ibv.at[c]], bbs[b], gs[b])

        def combine(b):
            ba, bb = bas[b], bbs[b]

            def row(r, carry):
                for cc in range(EMB // 16):
                    sl = pl.ds(cc * 16, 16)
                    if subtract:
                        ba[r, sl] = ba[r, sl] - bb[r, sl]
                    else:
                        ba[r, sl] = ba[r, sl] + bb[r, sl]
                return carry

            lax.fori_loop(0, KG, row, 0, unroll=4)

        for c0 in range(ahead):
            fire(c0, c0)

        def step(t, carry):
            for b in range(nslot):
                c = nslot * t + b
                bp = (b + ahead) % nslot

                @pl.when(c + ahead < NCH)
                def _():
                    @pl.when(c + ahead >= nslot)
                    def _():
                        pltpu.make_async_copy(
                            bas[bp], o.at[pl.ds(base, KG)], sts[bp]).wait()
                    fire(c + ahead, bp)

                pltpu.make_async_copy(ta.at[iav.at[0]], bas[b], gs[b]).wait()
                pltpu.make_async_copy(tb.at[ibv.at[0]], bbs[b], gs[b]).wait()
                combine(b)
                pltpu.async_copy(bas[b], o.at[pl.ds(base + c * KG, KG)], sts[b])
            return carry

        lax.fori_loop(0, NCH // nslot, step, 0)
        for b in range(nslot):
            pltpu.make_async_copy(bas[b], o.at[pl.ds(base, KG)], sts[b]).wait()

    return gathc


def _build_scatter(ne):
    EPW = ne // NW
    NCHS = EPW // KS
    nslot = 5
    ahead = 1
    assert NCHS % nslot == 0

    @functools.partial(
        pl.kernel,
        mesh=plsc.VectorSubcoreMesh(core_axis_name="c", subcore_axis_name="s"),
        out_type=jax.ShapeDtypeStruct((2 * NP, EMB), jnp.float32),
        scratch_types=[
            pltpu.VMEM((NCHS, KS), jnp.int32),
            pltpu.VMEM_SHARED((NP, EMB), jnp.float32),
        ]
        + [pltpu.VMEM((KS, EMB), jnp.float32)] * nslot
        + [pltpu.SemaphoreType.DMA] * (2 * nslot),
    )
    def scatter(vals, idx3, init, out, idxv, acc, *bufsem):
        bufs = bufsem[0:nslot]
        rds = bufsem[nslot:2 * nslot]
        scs = bufsem[2 * nslot:3 * nslot]
        ci = lax.axis_index("c")
        si = lax.axis_index("s")
        wid = si * 2 + ci
        base = wid * EPW
        pltpu.sync_copy(init.at[pl.ds(ci * NP + si * RPT, RPT)],
                        acc.at[pl.ds(si * RPT, RPT)])
        pltpu.sync_copy(idx3.at[wid], idxv)
        plsc.subcore_barrier()

        def fire_read(c, b):
            pltpu.async_copy(vals.at[pl.ds(base + c * KS, KS)], bufs[b], rds[b])

        for c0 in range(ahead):
            fire_read(c0, c0)

        def step(t, carry):
            for b in range(nslot):
                c = nslot * t + b
                bp = (b + ahead) % nslot

                @pl.when(c + ahead < NCHS)
                def _():
                    @pl.when(c + ahead >= nslot)
                    def _():
                        pltpu.make_async_copy(
                            bufs[bp], acc.at[idxv.at[0]], scs[bp]).wait()
                    fire_read(c + ahead, bp)

                pltpu.make_async_copy(
                    vals.at[pl.ds(base, KS)], bufs[b], rds[b]).wait()
                pltpu.async_copy(bufs[b], acc.at[idxv.at[c]], scs[b], add=True)
            return carry

        lax.fori_loop(0, NCHS // nslot, step, 0)
        for b in range(nslot):
            pltpu.make_async_copy(bufs[b], acc.at[idxv.at[0]], scs[b]).wait()
        plsc.subcore_barrier()
        pltpu.sync_copy(acc.at[pl.ds(si * RPT, RPT)],
                        out.at[pl.ds(ci * NP + si * RPT, RPT)])

    return scatter


_SC_CACHE = {}


def _sc_kernels():
    if not _SC_CACHE:
        _SC_CACHE["gadd"] = _build_gather_combine(subtract=False, ne=NEH)
        _SC_CACHE["gsub"] = _build_gather_combine(subtract=True, ne=NEH)
        _SC_CACHE["scat"] = _build_scatter(ne=NEH)
    return _SC_CACHE["gadd"], _SC_CACHE["gsub"], _SC_CACHE["scat"]


def _ln(h, g, b):
    mu = jnp.mean(h, axis=-1, keepdims=True)
    var = jnp.mean((h - mu) * (h - mu), axis=-1, keepdims=True)
    return (h - mu) * lax.rsqrt(var + 1e-5) * g + b


def _dot(a, b):
    return jnp.dot(a, b, preferred_element_type=jnp.float32)


BLK_E = 2000
BLK_N = 2000

_row = lambda i: (i, 0)
_fix = lambda i: (0, 0)


def _wspec():
    return pl.BlockSpec((EMB, EMB), _fix)


def _bspec():
    return pl.BlockSpec((1, EMB), _fix)


def _edge_body(e, g, w1, b1, w2, b2, w3, b3, lg, lb, o):
    x = e[...]
    h = _dot(x, w1[...]) + g[...] + b1[...]
    h = jnp.maximum(h, 0.0)
    h = jnp.maximum(_dot(h, w2[...]) + b2[...], 0.0)
    h = _dot(h, w3[...]) + b3[...]
    o[...] = x + _ln(h, lg[...], lb[...])


_EDGE_CALL = pl.pallas_call(
    _edge_body,
    grid=(NEH // BLK_E,),
    in_specs=[pl.BlockSpec((BLK_E, EMB), _row)] * 2
    + [_wspec(), _bspec(), _wspec(), _bspec(), _wspec(), _bspec(), _bspec(), _bspec()],
    out_specs=pl.BlockSpec((BLK_E, EMB), _row),
    out_shape=jax.ShapeDtypeStruct((NEH, EMB), jnp.float32),
)


def _node_body_emit(p, n, wa, wb, b1, w2, b2, w3, b3, lg, lb, wr, ws,
                    on, ob, oc):
    agg = p[0] + p[1]
    x = n[...]
    h = _dot(agg, wa[...]) + _dot(x, wb[...]) + b1[...]
    h = jnp.maximum(h, 0.0)
    h = jnp.maximum(_dot(h, w2[...]) + b2[...], 0.0)
    h = _dot(h, w3[...]) + b3[...]
    nn_ = x + _ln(h, lg[...], lb[...])
    on[...] = nn_
    ob[...] = _dot(nn_, wr[...])
    oc[...] = _dot(nn_, ws[...])


def _node_body_last(p, n, wa, wb, b1, w2, b2, w3, b3, lg, lb, on):
    agg = p[0] + p[1]
    x = n[...]
    h = _dot(agg, wa[...]) + _dot(x, wb[...]) + b1[...]
    h = jnp.maximum(h, 0.0)
    h = jnp.maximum(_dot(h, w2[...]) + b2[...], 0.0)
    h = _dot(h, w3[...]) + b3[...]
    on[...] = x + _ln(h, lg[...], lb[...])


_node_in_specs = [
    pl.BlockSpec((2, BLK_N, EMB), lambda i: (0, i, 0)),
    pl.BlockSpec((BLK_N, EMB), _row),
    _wspec(), _wspec(), _bspec(), _wspec(), _bspec(), _wspec(), _bspec(),
    _bspec(), _bspec(),
]

_NODE_CALL_EMIT = pl.pallas_call(
    _node_body_emit,
    grid=(NN // BLK_N,),
    in_specs=_node_in_specs + [_wspec(), _wspec()],
    out_specs=[pl.BlockSpec((BLK_N, EMB), _row)] * 3,
    out_shape=[jax.ShapeDtypeStruct((NN, EMB), jnp.float32)] * 3,
)

_NODE_CALL_LAST = pl.pallas_call(
    _node_body_last,
    grid=(NN // BLK_N,),
    in_specs=_node_in_specs,
    out_specs=pl.BlockSpec((BLK_N, EMB), _row),
    out_shape=jax.ShapeDtypeStruct((NN, EMB), jnp.float32),
)


def _enc_node_body(x, w1, b1, w2, b2, w3, b3, lg, lb, wr, ws, on, ob, oc):
    h = jnp.maximum(_dot(x[...], w1[...]) + b1[...], 0.0)
    h = jnp.maximum(_dot(h, w2[...]) + b2[...], 0.0)
    h = _dot(h, w3[...]) + b3[...]
    nn_ = _ln(h, lg[...], lb[...])
    on[...] = nn_
    ob[...] = _dot(nn_, wr[...])
    oc[...] = _dot(nn_, ws[...])


_ENC_NODE_CALL = pl.pallas_call(
    _enc_node_body,
    grid=(NN // BLK_N,),
    in_specs=[pl.BlockSpec((BLK_N, 24), _row),
              pl.BlockSpec((24, EMB), _fix), _bspec(),
              _wspec(), _bspec(), _wspec(), _bspec(), _bspec(), _bspec(),
              _wspec(), _wspec()],
    out_specs=[pl.BlockSpec((BLK_N, EMB), _row)] * 3,
    out_shape=[jax.ShapeDtypeStruct((NN, EMB), jnp.float32)] * 3,
)


def _enc_edge_body(rel_ref, w1p, w1d, b1, w2, b2, w3, b3, lg, lb, o):
    rel = rel_ref[...]
    dist = jnp.sqrt(jnp.sum(rel * rel, axis=-1, keepdims=True))
    h = _dot(rel, w1p[...]) + dist * w1d[...] + b1[...]
    h = jnp.maximum(h, 0.0)
    h = jnp.maximum(_dot(h, w2[...]) + b2[...], 0.0)
    h = _dot(h, w3[...]) + b3[...]
    o[...] = _ln(h, lg[...], lb[...])


_ENC_EDGE_CALL = pl.pallas_call(
    _enc_edge_body,
    grid=(NEH // BLK_E,),
    in_specs=[pl.BlockSpec((BLK_E, EMB), _row)]
    + [_wspec(), _bspec(), _bspec(),
       _wspec(), _bspec(), _wspec(), _bspec(), _bspec(), _bspec()],
    out_specs=pl.BlockSpec((BLK_E, EMB), _row),
    out_shape=jax.ShapeDtypeStruct((NEH, EMB), jnp.float32),
)


def _dec_body(n, w1, b1, w2, b2, w3, b3, o):
    h = jnp.maximum(_dot(n[...], w1[...]) + b1[...], 0.0)
    h = jnp.maximum(_dot(h, w2[...]) + b2[...], 0.0)
    o[...] = _dot(h, w3[...]) + b3[...]


_DEC_CALL = pl.pallas_call(
    _dec_body,
    grid=(NN // BLK_N,),
    in_specs=[pl.BlockSpec((BLK_N, EMB), _row),
              _wspec(), _bspec(), _wspec(), _bspec(), _wspec(), _bspec()],
    out_specs=pl.BlockSpec((BLK_N, EMB), _row),
    out_shape=jax.ShapeDtypeStruct((NN, EMB), jnp.float32),
)


def _r(b):
    return b.reshape(1, EMB)


def kernel(velocities, positions, params, materials, neighbor_idxs):
    recv = neighbor_idxs[:, 0].astype(jnp.int32)
    send = neighbor_idxs[:, 1].astype(jnp.int32)

    Wm, bm = params["mat_enc"]
    (w1, b1), (w2, b2), (w3, b3) = params["node_enc"]
    enc_w1 = jnp.concatenate([w1[:15], Wm @ w1[15:]], axis=0)
    enc_b1 = b1 + bm @ w1[15:]
    lng0, lnb0 = params["node_enc_ln"]

    (ew1, eb1), (ew2, eb2), (ew3, eb3) = params["edge_enc"]
    ew1p = jnp.zeros((EMB, EMB), jnp.float32).at[:3].set(ew1[:3])
    ew1d = ew1[3:4]
    elng, elnb = params["edge_enc_ln"]

    pe = []
    for i in range(NLAYERS):
        (a1, c1), (a2, c2), (a3, c3) = params["proc_edge"][i]
        pe.append((a1[:EMB], a1[EMB:2 * EMB], a1[2 * EMB:], c1, a2, c2, a3, c3))
    pn = []
    for i in range(NLAYERS):
        (a1, c1), (a2, c2), (a3, c3) = params["proc_node"][i]
        pn.append((a1[:EMB], a1[EMB:], c1, a2, c2, a3, c3))

    x_feat = jnp.concatenate(
        [velocities.reshape(NN, -1),
         jax.nn.one_hot(materials, 9, dtype=jnp.float32)], axis=1)
    pos128 = jnp.zeros((NN, EMB), jnp.float32).at[:, :3].set(positions)
    nchh = (NEH // NW) // KG
    recv3a = recv[:NEH].reshape(NW, nchh, KG)
    recv3b = recv[NEH:].reshape(NW, nchh, KG)
    send3a = send[:NEH].reshape(NW, nchh, KG)
    send3b = send[NEH:].reshape(NW, nchh, KG)
    zeros = jnp.zeros((2 * NP, EMB), jnp.float32)

    gadd, gsub, scat = _sc_kernels()

    nodes, tb, tc = _ENC_NODE_CALL(x_feat, enc_w1, _r(enc_b1), w2, _r(b2),
                                   w3, _r(b3), _r(lng0), _r(lnb0),
                                   pe[0][1], pe[0][2])
    rela = gsub(pos128, pos128, recv3a, send3a)
    relb = gsub(pos128, pos128, recv3b, send3b)
    ea = _ENC_EDGE_CALL(rela, ew1p, ew1d, _r(eb1), ew2, _r(eb2),
                        ew3, _r(eb3), _r(elng), _r(elnb))
    eb = _ENC_EDGE_CALL(relb, ew1p, ew1d, _r(eb1), ew2, _r(eb2),
                        ew3, _r(eb3), _r(elng), _r(elnb))

    for i in range(NLAYERS):
        w1e, _, _, c1, a2, c2, a3, c3 = pe[i]
        lg, lb = params["proc_edge_ln"][i]
        ga = gadd(tb, tc, recv3a, send3a)
        gb = gadd(tb, tc, recv3b, send3b)
        ea = _EDGE_CALL(ea, ga, w1e, _r(c1), a2, _r(c2),
                        a3, _r(c3), _r(lg), _r(lb))
        pa = scat(ea, recv3a, zeros)
        eb = _EDGE_CALL(eb, gb, w1e, _r(c1), a2, _r(c2),
                        a3, _r(c3), _r(lg), _r(lb))
        parts = scat(eb, recv3b, pa).reshape(2, NP, EMB)[:, :NN]
        na, nb, d1, n2, d2, n3, d3 = pn[i]
        nlg, nlb = params["proc_node_ln"][i]
        if i < NLAYERS - 1:
            nodes, tb, tc = _NODE_CALL_EMIT(
                parts, nodes, na, nb, _r(d1), n2, _r(d2), n3, _r(d3),
                _r(nlg), _r(nlb), pe[i + 1][1], pe[i + 1][2])
        else:
            nodes = _NODE_CALL_LAST(
                parts, nodes, na, nb, _r(d1), n2, _r(d2), n3, _r(d3),
                _r(nlg), _r(nlb))

    (dw1, db1), (dw2, db2), (dw3, db3) = params["dec"]
    dw3p = jnp.zeros((EMB, EMB), jnp.float32).at[:, :3].set(dw3)
    db3p = jnp.zeros((EMB,), jnp.float32).at[:3].set(db3)
    out = _DEC_CALL(nodes, dw1, _r(db1), dw2, _r(db2), dw3p, _r(db3p))
    return out[:, :3]
